# initial kernel scaffold (unmeasured)
import jax
import jax.numpy as jnp
from jax import lax
from jax.experimental import pallas as pl
from jax.experimental.pallas import tpu as pltpu

B, QL, H, D = 8, 1, 8, 64
P_SHARD = 64
BS = 16
NK = P_SHARD * BS
NBT = 64
SCALE = D ** -0.5
NEG = -1e30


def kernel(Q, K, V, bt, lens):
    lens2 = lens.reshape(B, 1)

    def body(q_ref, k_ref, v_ref, bt_ref, lens_ref, out_ref,
             acc_s, acc_r, st_s, st_r,
             acc_send_sem, acc_recv_sem, st_send_sem, st_recv_sem):
        my_x = lax.axis_index("x")
        my_y = lax.axis_index("y")
        peer = (my_x, 1 - my_y)

        barrier = pltpu.get_barrier_semaphore()
        pl.semaphore_signal(
            barrier, inc=1, device_id=peer,
            device_id_type=pl.DeviceIdType.MESH,
        )
        pl.semaphore_wait(barrier, 1)

        btv = bt_ref[...]
        lensv = lens_ref[...]
        offset = my_y * P_SHARD
        jmask = lax.broadcasted_iota(jnp.int32, (B, NBT), 1) < lensv
        lp = btv - offset
        pid = lax.broadcasted_iota(jnp.int32, (B, NBT, P_SHARD), 2)
        hit = (lp[:, :, None] == pid) & jmask[:, :, None]
        cnt = jnp.sum(hit.astype(jnp.float32), axis=1)
        w = jnp.broadcast_to(cnt[:, :, None], (B, P_SHARD, BS))
        w = w.reshape(B, NK)

        q = q_ref[...]

        for h in range(H):
            qh = q[:, 0, h, :]
            kh = k_ref[:, :, h, :].reshape(NK, D)
            vh = v_ref[:, :, h, :].reshape(NK, D)
            s = lax.dot_general(
                qh, kh, (((1,), (1,)), ((), ())),
                preferred_element_type=jnp.float32,
            ) * SCALE
            s = jnp.where(w > 0, s, NEG)
            m = jnp.max(s, axis=1, keepdims=True)
            p = w * jnp.exp(s - m)
            l = jnp.sum(p, axis=1, keepdims=True)
            acc = lax.dot_general(
                p, vh, (((1,), (0,)), ((), ())),
                preferred_element_type=jnp.float32,
            )
            acc_s[h] = acc
            st_s[h] = jnp.concatenate([m, l], axis=1)

        rdma_acc = pltpu.make_async_remote_copy(
            src_ref=acc_s, dst_ref=acc_r,
            send_sem=acc_send_sem, recv_sem=acc_recv_sem,
            device_id=peer, device_id_type=pl.DeviceIdType.MESH,
        )
        rdma_st = pltpu.make_async_remote_copy(
            src_ref=st_s, dst_ref=st_r,
            send_sem=st_send_sem, recv_sem=st_recv_sem,
            device_id=peer, device_id_type=pl.DeviceIdType.MESH,
        )
        rdma_acc.start()
        rdma_st.start()
        rdma_acc.wait()
        rdma_st.wait()

        for h in range(H):
            stl = st_s[h]
            strr = st_r[h]
            m_l, l_l = stl[:, 0:1], stl[:, 1:2]
            m_r, l_r = strr[:, 0:1], strr[:, 1:2]
            mn = jnp.maximum(m_l, m_r)
            a = jnp.exp(m_l - mn)
            b = jnp.exp(m_r - mn)
            ln = a * l_l + b * l_r
            o = (a * acc_s[h] + b * acc_r[h]) / ln
            out_ref[:, 0, h, :] = o

    out_shape = jax.ShapeDtypeStruct((B, QL, H, D), jnp.float32)
    return pl.pallas_call(
        body,
        out_shape=out_shape,
        in_specs=[pl.BlockSpec(memory_space=pltpu.VMEM)] * 5,
        out_specs=pl.BlockSpec(memory_space=pltpu.VMEM),
        scratch_shapes=[
            pltpu.VMEM((H, B, D), jnp.float32),
            pltpu.VMEM((H, B, D), jnp.float32),
            pltpu.VMEM((H, B, 2), jnp.float32),
            pltpu.VMEM((H, B, 2), jnp.float32),
            pltpu.SemaphoreType.DMA,
            pltpu.SemaphoreType.DMA,
            pltpu.SemaphoreType.DMA,
            pltpu.SemaphoreType.DMA,
        ],
        compiler_params=pltpu.CompilerParams(collective_id=0),
    )(Q, K, V, bt, lens2)


# baseline (device time: 16858 ns/iter reference)
import jax
import jax.numpy as jnp
from jax import lax
from jax.experimental import pallas as pl
from jax.experimental.pallas import tpu as pltpu

B, QL, H, D = 8, 1, 8, 64
P_SHARD = 64
BS = 16
NK = P_SHARD * BS
NBT = 64
SCALE = D ** -0.5
NEG = -1e30


def kernel(Q, K, V, bt, lens):
    lens2 = lens.reshape(B, 1)

    def body(q_ref, k_ref, v_ref, bt_ref, lens_ref, out_ref,
             acc_s, acc_r, st_s, st_r,
             acc_send_sem, acc_recv_sem, st_send_sem, st_recv_sem):
        my_x = lax.axis_index("x")
        my_y = lax.axis_index("y")
        peer = (my_x, 1 - my_y)

        barrier = pltpu.get_barrier_semaphore()
        pl.semaphore_signal(
            barrier, inc=1, device_id=peer,
            device_id_type=pl.DeviceIdType.MESH,
        )
        pl.semaphore_wait(barrier, 1)

        btv = bt_ref[...]
        lensv = lens_ref[...]
        offset = my_y * P_SHARD
        jmask = lax.broadcasted_iota(jnp.int32, (B, NBT), 1) < lensv
        lp = jnp.where(jmask, btv - offset, -1)
        pid = lax.broadcasted_iota(jnp.int32, (B, P_SHARD), 1)
        cnt = jnp.zeros((B, P_SHARD), jnp.float32)
        for j in range(NBT):
            cnt = cnt + (lp[:, j:j + 1] == pid).astype(jnp.float32)
        rowi = lax.broadcasted_iota(jnp.int32, (P_SHARD, NK), 0)
        coli = lax.broadcasted_iota(jnp.int32, (P_SHARD, NK), 1)
        expand = (coli // BS == rowi).astype(jnp.float32)
        w = lax.dot_general(
            cnt, expand, (((1,), (0,)), ((), ())),
            preferred_element_type=jnp.float32,
        )

        q = q_ref[...]

        for h in range(H):
            qh = q[:, 0, h, :]
            kh = k_ref[:, :, h, :].reshape(NK, D)
            vh = v_ref[:, :, h, :].reshape(NK, D)
            s = lax.dot_general(
                qh, kh, (((1,), (1,)), ((), ())),
                preferred_element_type=jnp.float32,
            ) * SCALE
            s = jnp.where(w > 0, s, NEG)
            m = jnp.max(s, axis=1, keepdims=True)
            p = w * jnp.exp(s - m)
            l = jnp.sum(p, axis=1, keepdims=True)
            acc = lax.dot_general(
                p, vh, (((1,), (0,)), ((), ())),
                preferred_element_type=jnp.float32,
            )
            acc_s[h] = acc
            st_s[h] = jnp.concatenate([m, l], axis=1)

        rdma_acc = pltpu.make_async_remote_copy(
            src_ref=acc_s, dst_ref=acc_r,
            send_sem=acc_send_sem, recv_sem=acc_recv_sem,
            device_id=peer, device_id_type=pl.DeviceIdType.MESH,
        )
        rdma_st = pltpu.make_async_remote_copy(
            src_ref=st_s, dst_ref=st_r,
            send_sem=st_send_sem, recv_sem=st_recv_sem,
            device_id=peer, device_id_type=pl.DeviceIdType.MESH,
        )
        rdma_acc.start()
        rdma_st.start()
        rdma_acc.wait()
        rdma_st.wait()

        for h in range(H):
            stl = st_s[h]
            strr = st_r[h]
            m_l, l_l = stl[:, 0:1], stl[:, 1:2]
            m_r, l_r = strr[:, 0:1], strr[:, 1:2]
            mn = jnp.maximum(m_l, m_r)
            a = jnp.exp(m_l - mn)
            b = jnp.exp(m_r - mn)
            ln = a * l_l + b * l_r
            o = (a * acc_s[h] + b * acc_r[h]) / ln
            out_ref[:, 0, h, :] = o

    out_shape = jax.ShapeDtypeStruct((B, QL, H, D), jnp.float32)
    return pl.pallas_call(
        body,
        out_shape=out_shape,
        in_specs=[pl.BlockSpec(memory_space=pltpu.VMEM)] * 5,
        out_specs=pl.BlockSpec(memory_space=pltpu.VMEM),
        scratch_shapes=[
            pltpu.VMEM((H, B, D), jnp.float32),
            pltpu.VMEM((H, B, D), jnp.float32),
            pltpu.VMEM((H, B, 2), jnp.float32),
            pltpu.VMEM((H, B, 2), jnp.float32),
            pltpu.SemaphoreType.DMA,
            pltpu.SemaphoreType.DMA,
            pltpu.SemaphoreType.DMA,
            pltpu.SemaphoreType.DMA,
        ],
        compiler_params=pltpu.CompilerParams(collective_id=0),
    )(Q, K, V, bt, lens2)


# device time: 16655 ns/iter; 1.0122x vs baseline; 1.0122x over previous
import jax
import jax.numpy as jnp
from jax import lax
from jax.experimental import pallas as pl
from jax.experimental.pallas import tpu as pltpu

B, QL, H, D = 8, 1, 8, 64
P_SHARD = 64
BS = 16
NK = P_SHARD * BS
NBT = 64
SCALE = D ** -0.5
NEG = -1e30


def kernel(Q, K, V, bt, lens):
    lens2 = lens.reshape(B, 1)

    def body(q_ref, k_ref, v_ref, bt_ref, lens_ref, out_ref,
             acc_s, acc_r, st_s, st_r,
             acc_send_sem, acc_recv_sem, st_send_sem, st_recv_sem):
        my_x = lax.axis_index("x")
        my_y = lax.axis_index("y")
        peer = (my_x, 1 - my_y)

        barrier = pltpu.get_barrier_semaphore()
        pl.semaphore_signal(
            barrier, inc=1, device_id=peer,
            device_id_type=pl.DeviceIdType.MESH,
        )
        pl.semaphore_wait(barrier, 1)

        btv = bt_ref[...]
        lensv = lens_ref[...]
        offset = my_y * P_SHARD
        jmask = lax.broadcasted_iota(jnp.int32, (B, NBT), 1) < lensv
        lp = jnp.where(jmask, btv - offset, -1)
        pid = lax.broadcasted_iota(jnp.int32, (B, P_SHARD), 1)
        cnt = jnp.zeros((B, P_SHARD), jnp.float32)
        for j in range(NBT):
            cnt = cnt + (lp[:, j:j + 1] == pid).astype(jnp.float32)
        rowi = lax.broadcasted_iota(jnp.int32, (P_SHARD, NK), 0)
        coli = lax.broadcasted_iota(jnp.int32, (P_SHARD, NK), 1)
        expand = (coli // BS == rowi).astype(jnp.float32)
        w = lax.dot_general(
            cnt, expand, (((1,), (0,)), ((), ())),
            preferred_element_type=jnp.float32,
        )

        q = q_ref[...]

        def compute_head(h):
            qh = q[:, 0, h, :]
            kh = k_ref[:, :, h, :].reshape(NK, D)
            vh = v_ref[:, :, h, :].reshape(NK, D)
            s = lax.dot_general(
                qh, kh, (((1,), (1,)), ((), ())),
                preferred_element_type=jnp.float32,
            ) * SCALE
            s = jnp.where(w > 0, s, NEG)
            m = jnp.max(s, axis=1, keepdims=True)
            p = w * jnp.exp(s - m)
            l = jnp.sum(p, axis=1, keepdims=True)
            acc = lax.dot_general(
                p, vh, (((1,), (0,)), ((), ())),
                preferred_element_type=jnp.float32,
            )
            acc_s[h] = acc
            st_s[h] = jnp.concatenate([m, l], axis=1)

        HG = H // 2

        def wave_rdmas(g):
            sl = pl.ds(g * HG, HG)
            return (
                pltpu.make_async_remote_copy(
                    src_ref=acc_s.at[sl], dst_ref=acc_r.at[sl],
                    send_sem=acc_send_sem.at[g], recv_sem=acc_recv_sem.at[g],
                    device_id=peer, device_id_type=pl.DeviceIdType.MESH,
                ),
                pltpu.make_async_remote_copy(
                    src_ref=st_s.at[sl], dst_ref=st_r.at[sl],
                    send_sem=st_send_sem.at[g], recv_sem=st_recv_sem.at[g],
                    device_id=peer, device_id_type=pl.DeviceIdType.MESH,
                ),
            )

        def combine_head(h):
            stl = st_s[h]
            strr = st_r[h]
            m_l, l_l = stl[:, 0:1], stl[:, 1:2]
            m_r, l_r = strr[:, 0:1], strr[:, 1:2]
            mn = jnp.maximum(m_l, m_r)
            a = jnp.exp(m_l - mn)
            b = jnp.exp(m_r - mn)
            ln = a * l_l + b * l_r
            o = (a * acc_s[h] + b * acc_r[h]) / ln
            out_ref[:, 0, h, :] = o

        for h in range(HG):
            compute_head(h)
        acc0, st0 = wave_rdmas(0)
        acc0.start()
        st0.start()
        for h in range(HG, H):
            compute_head(h)
        acc1, st1 = wave_rdmas(1)
        acc1.start()
        st1.start()
        acc0.wait_recv()
        st0.wait_recv()
        for h in range(HG):
            combine_head(h)
        acc1.wait_recv()
        st1.wait_recv()
        for h in range(HG, H):
            combine_head(h)
        for r in (acc0, st0, acc1, st1):
            r.wait_send()

    out_shape = jax.ShapeDtypeStruct((B, QL, H, D), jnp.float32)
    return pl.pallas_call(
        body,
        out_shape=out_shape,
        in_specs=[pl.BlockSpec(memory_space=pltpu.VMEM)] * 5,
        out_specs=pl.BlockSpec(memory_space=pltpu.VMEM),
        scratch_shapes=[
            pltpu.VMEM((H, B, D), jnp.float32),
            pltpu.VMEM((H, B, D), jnp.float32),
            pltpu.VMEM((H, B, 2), jnp.float32),
            pltpu.VMEM((H, B, 2), jnp.float32),
            pltpu.SemaphoreType.DMA((2,)),
            pltpu.SemaphoreType.DMA((2,)),
            pltpu.SemaphoreType.DMA((2,)),
            pltpu.SemaphoreType.DMA((2,)),
        ],
        compiler_params=pltpu.CompilerParams(collective_id=0),
    )(Q, K, V, bt, lens2)
